# baseline (speedup 1.0000x reference)
import functools

import jax
import jax.numpy as jnp
from jax import lax
from jax.experimental import pallas as pl
from jax.experimental.pallas import tpu as pltpu
from jax.experimental.pallas import tpu_sc as plsc

H = 128
NBASES = 8
RPAD = 512
LANES = 16
NC, NS = 2, 16
NW = NC * NS
CHUNK = 80


def _sc_mesh():
    return plsc.VectorSubcoreMesh(core_axis_name="c", subcore_axis_name="s")


def _sc_gather_h0(N):
    n_chunks = N // CHUNK
    n_iters = -(-n_chunks // NW)

    @functools.partial(
        pl.kernel,
        mesh=_sc_mesh(),
        out_type=jax.ShapeDtypeStruct((N, H), jnp.float32),
        scratch_types=[
            pltpu.VMEM((CHUNK,), jnp.int32),
            pltpu.VMEM((CHUNK, H), jnp.float32),
            pltpu.SemaphoreType.DMA,
        ],
    )
    def k(ent_hbm, nid_hbm, h0_hbm, idx_v, rows_v, sem):
        wid = lax.axis_index("s") * NC + lax.axis_index("c")

        def body(kk, _):
            cidx = wid + kk * NW

            @pl.when(cidx < n_chunks)
            def _go():
                base = cidx * CHUNK
                pltpu.sync_copy(nid_hbm.at[pl.ds(base, CHUNK)], idx_v)
                pltpu.async_copy(ent_hbm.at[idx_v], rows_v, sem).wait()
                pltpu.sync_copy(rows_v, h0_hbm.at[pl.ds(base, CHUNK)])

            return _

        lax.fori_loop(0, n_iters, body, 0)

    return k


def _sc_gather_rows(N, E):
    CH = 128
    NBUF = 4
    nch = E // CH
    cpt = nch // NW
    extra = nch - cpt * NW
    ngrp = -(-(cpt + 1) // NBUF)

    @functools.partial(
        pl.kernel,
        mesh=_sc_mesh(),
        out_type=jax.ShapeDtypeStruct((E, H), jnp.float32),
        scratch_types=[
            pltpu.VMEM((cpt * CH + CH,), jnp.int32),
            [pltpu.VMEM((CH, H), jnp.float32) for _ in range(NBUF)],
            [pltpu.SemaphoreType.DMA for _ in range(NBUF)],
            [pltpu.SemaphoreType.DMA for _ in range(NBUF)],
        ],
    )
    def k(h_hbm, src_hbm, out_hbm, idx_all, rows, gsem, wsem):
        wid = lax.axis_index("s") * NC + lax.axis_index("c")
        nt = cpt + (wid < extra).astype(jnp.int32)
        pltpu.sync_copy(src_hbm.at[pl.ds(wid * (cpt * CH), cpt * CH)],
                        idx_all.at[pl.ds(0, cpt * CH)])

        @pl.when(wid < extra)
        def _extra():
            pltpu.sync_copy(src_hbm.at[pl.ds((cpt * NW + wid) * CH, CH)],
                            idx_all.at[pl.ds(cpt * CH, CH)])

        def out_base(kk):
            return jnp.where(kk < cpt, (wid * cpt + kk) * CH,
                             (cpt * NW + wid) * CH)

        for b in range(NBUF):
            pltpu.async_copy(h_hbm.at[idx_all.at[pl.ds(b * CH, CH)]],
                             rows[b], gsem[b])

        def grp(g, _):
            for b in range(NBUF):
                kk = g * NBUF + b

                @pl.when(kk < nt)
                def _do(kk=kk, b=b):
                    pltpu.make_async_copy(
                        h_hbm.at[idx_all.at[pl.ds(0, CH)]], rows[b],
                        gsem[b]).wait()
                    pltpu.async_copy(rows[b],
                                     out_hbm.at[pl.ds(out_base(kk), CH)],
                                     wsem[b])

                kn = kk + NBUF

                @pl.when(kn < nt)
                def _next(kn=kn, b=b):
                    pltpu.make_async_copy(
                        rows[b], out_hbm.at[pl.ds(0, CH)], wsem[b]).wait()
                    pltpu.async_copy(
                        h_hbm.at[idx_all.at[pl.ds(kn * CH, CH)]],
                        rows[b], gsem[b])

            return _

        lax.fori_loop(0, ngrp, grp, 0)
        for b in range(NBUF):
            pltpu.make_async_copy(rows[b], out_hbm.at[pl.ds(0, CH)],
                                  wsem[b]).wait()

    return k


def _sc_scatter_add(N, E):
    NH = N // NC
    CH = 128
    NBUF = 3
    DPAD = 8
    EH = E // 2
    nch = EH // CH
    cpt = nch // NS
    extra = nch - cpt * NS
    ngrp = -(-(cpt + 1) // NBUF)
    ZCH = 80
    n_chunks_z = NH // ZCH
    z_iters = -(-n_chunks_z // NS)

    @functools.partial(
        pl.kernel,
        mesh=_sc_mesh(),
        out_type=jax.ShapeDtypeStruct((N, H), jnp.float32),
        scratch_types=[
            pltpu.VMEM_SHARED((NH + DPAD, H), jnp.float32),
            pltpu.VMEM((NBUF, CH), jnp.int32),
            pltpu.VMEM((NBUF, CH), jnp.int32),
            [pltpu.VMEM((CH, H), jnp.float32) for _ in range(NBUF)],
            [pltpu.SemaphoreType.DMA for _ in range(NBUF)],
            [pltpu.SemaphoreType.DMA for _ in range(NBUF)],
        ],
    )
    def k(msga_hbm, msgb_hbm, dst_hbm, agg_hbm, aggS, dst2d, li2d, mbufs,
          gsem, dsem):
        c = lax.axis_index("c")
        s = lax.axis_index("s")
        node0 = c * NH
        nt = cpt + (s < extra).astype(jnp.int32)

        def msg_base(kk):
            return jnp.where(kk < cpt, (s * cpt + kk) * CH,
                             (cpt * NS + s) * CH)

        zero = jnp.zeros((LANES,), jnp.float32)

        def zb(i, _):
            for j in range(H // LANES):
                mbufs[0][i, pl.ds(j * LANES, LANES)] = zero
            return _

        lax.fori_loop(0, CH, zb, 0)

        def zs(kk, _):
            cidx = s + kk * NS

            @pl.when(cidx < n_chunks_z)
            def _go():
                pltpu.sync_copy(mbufs[0].at[pl.ds(0, ZCH)],
                                aggS.at[pl.ds(cidx * ZCH, ZCH)])

            return _

        lax.fori_loop(0, z_iters, zs, 0)

        @pl.when(s == 0)
        def _zdummy():
            pltpu.sync_copy(mbufs[0].at[pl.ds(0, DPAD)],
                            aggS.at[pl.ds(NH, DPAD)])

        plsc.subcore_barrier()

        def run_half(msg_hbm, hoff):
            def start_chunk(kk, b):
                base = msg_base(kk)
                pltpu.async_copy(dst_hbm.at[pl.ds(hoff + base, CH)],
                                 dst2d.at[b], dsem[b])
                pltpu.async_copy(msg_hbm.at[pl.ds(base, CH)], mbufs[b],
                                 gsem[b])

            for b in range(NBUF):
                start_chunk(b, b)

            def grp(g, _):
                for b in range(NBUF):
                    kk = g * NBUF + b

                    @pl.when(kk < nt)
                    def _do(kk=kk, b=b):
                        pltpu.make_async_copy(
                            dst_hbm.at[pl.ds(0, CH)], dst2d.at[b],
                            dsem[b]).wait()
                        for j in range(CH // LANES):
                            sl = pl.ds(j * LANES, LANES)
                            li = dst2d[b, sl] - node0
                            oob = (li < 0) | (li >= NH)
                            li2d[b, sl] = jnp.where(oob, NH, li)
                        pltpu.make_async_copy(
                            msg_hbm.at[pl.ds(0, CH)], mbufs[b],
                            gsem[b]).wait()
                        pltpu.sync_copy(mbufs[b], aggS.at[li2d.at[b]],
                                        add=True)

                    kn = kk + NBUF

                    @pl.when(kn < nt)
                    def _next(kn=kn, b=b):
                        start_chunk(kn, b)

                return _

            lax.fori_loop(0, ngrp, grp, 0)

        run_half(msga_hbm, 0)
        run_half(msgb_hbm, EH)
        plsc.subcore_barrier()

        def wb(kk, _):
            cidx = s + kk * NS

            @pl.when(cidx < n_chunks_z)
            def _go():
                base = cidx * ZCH
                pltpu.sync_copy(aggS.at[pl.ds(base, ZCH)],
                                mbufs[1].at[pl.ds(0, ZCH)])
                pltpu.sync_copy(mbufs[1].at[pl.ds(0, ZCH)],
                                agg_hbm.at[pl.ds(node0 + base, ZCH)])

            return _

        lax.fori_loop(0, z_iters, wb, 0)

    return k


def _tc_msg(E, blk0=0):
    EB = 1600

    def body(hs_ref, et_ref, wc_ref, sel_ref, wall_ref, msg_ref):
        g = jnp.dot(hs_ref[...].astype(jnp.bfloat16),
                    wall_ref[...].astype(jnp.bfloat16),
                    preferred_element_type=jnp.float32)
        et = et_ref[...].astype(jnp.int32)
        onehot = (et == lax.broadcasted_iota(jnp.int32, (1, RPAD), 1)
                  ).astype(jnp.bfloat16)
        coef = jnp.dot(onehot, wc_ref[...].astype(jnp.bfloat16),
                       preferred_element_type=jnp.float32)
        coefe = jnp.dot(coef.astype(jnp.bfloat16), sel_ref[...],
                        preferred_element_type=jnp.float32)
        msg = coefe[:, 0:H] * g[:, 0:H]
        for b in range(1, NBASES):
            msg = msg + coefe[:, b * H:(b + 1) * H] * g[:, b * H:(b + 1) * H]
        msg_ref[...] = msg

    return pl.pallas_call(
        body,
        grid=(E // EB,),
        in_specs=[
            pl.BlockSpec((EB, H), lambda i: (i, 0)),
            pl.BlockSpec((EB, 1), lambda i: (i + blk0, 0)),
            pl.BlockSpec((RPAD, NBASES), lambda i: (0, 0)),
            pl.BlockSpec((NBASES, NBASES * H), lambda i: (0, 0)),
            pl.BlockSpec((H, NBASES * H), lambda i: (0, 0)),
        ],
        out_specs=pl.BlockSpec((EB, H), lambda i: (i, 0)),
        out_shape=jax.ShapeDtypeStruct((E, H), jnp.float32),
    )


def _tc_hnext(N):
    NBK = 2000

    def body(agg_ref, h_ref, wl_ref, out_ref):
        out = agg_ref[...] + jnp.dot(h_ref[...], wl_ref[...],
                                     preferred_element_type=jnp.float32)
        out_ref[...] = jnp.maximum(out, 0.0)

    return pl.pallas_call(
        body,
        grid=(N // NBK,),
        in_specs=[
            pl.BlockSpec((NBK, H), lambda i: (i, 0)),
            pl.BlockSpec((NBK, H), lambda i: (i, 0)),
            pl.BlockSpec((H, H), lambda i: (0, 0)),
        ],
        out_specs=pl.BlockSpec((NBK, H), lambda i: (i, 0)),
        out_shape=jax.ShapeDtypeStruct((N, H), jnp.float32),
    )


def _tc_pool(N, SEQ):
    NBK = 2000
    nsteps = N // NBK

    def body(agg_ref, h_ref, wl_ref, gid_ref, out_ref, acc_ref):
        i = pl.program_id(0)

        @pl.when(i == 0)
        def _init():
            acc_ref[...] = jnp.full((16, H), -jnp.inf, jnp.float32)

        h2 = agg_ref[...] + jnp.dot(h_ref[...], wl_ref[...],
                                    preferred_element_type=jnp.float32)
        gid = gid_ref[...]
        for g in range(SEQ):
            cand = jnp.where(gid == g, h2, -jnp.inf)
            acc_ref[g:g + 1, :] = jnp.maximum(
                acc_ref[g:g + 1, :], jnp.max(cand, axis=0, keepdims=True))

        @pl.when(i == nsteps - 1)
        def _fin():
            out_ref[...] = acc_ref[...]

    return pl.pallas_call(
        body,
        grid=(nsteps,),
        in_specs=[
            pl.BlockSpec((NBK, H), lambda i: (i, 0)),
            pl.BlockSpec((NBK, H), lambda i: (i, 0)),
            pl.BlockSpec((H, H), lambda i: (0, 0)),
            pl.BlockSpec((NBK, 1), lambda i: (i, 0)),
        ],
        out_specs=pl.BlockSpec((16, H), lambda i: (0, 0)),
        out_shape=jax.ShapeDtypeStruct((16, H), jnp.float32),
        scratch_shapes=[pltpu.VMEM((16, H), jnp.float32)],
    )


def _tc_select(TQ, SEQ):
    def body(gi_ref, tid_ref, out_ref):
        tid = tid_ref[...]
        acc = jnp.zeros((TQ, H), jnp.float32)
        for g in range(SEQ):
            acc = jnp.where(tid == g, gi_ref[g:g + 1, :], acc)
        out_ref[...] = acc

    return pl.pallas_call(
        body,
        grid=(1,),
        in_specs=[
            pl.BlockSpec((16, H), lambda i: (0, 0)),
            pl.BlockSpec((TQ, 1), lambda i: (0, 0)),
        ],
        out_specs=pl.BlockSpec((TQ, H), lambda i: (0, 0)),
        out_shape=jax.ShapeDtypeStruct((TQ, H), jnp.float32),
    )


def kernel(ent_embeds, node_id, src, dst, edge_type, graph_ids, time_idx,
           W_basis1, w_comp1, W_loop1, W_basis2, w_comp2, W_loop2):
    N = node_id.shape[0]
    E = src.shape[0]
    TB, SEQ = time_idx.shape

    EHALF = E // 2
    nid = node_id.astype(jnp.int32)
    srci = src.astype(jnp.int32)
    dsti = dst.astype(jnp.int32)
    et2d = edge_type.astype(jnp.int16).reshape(E, 1)
    nblk_half = EHALF // 1600

    wall1 = jnp.transpose(W_basis1, (1, 0, 2)).reshape(H, NBASES * H)
    wall2 = jnp.transpose(W_basis2, (1, 0, 2)).reshape(H, NBASES * H)
    nrel = w_comp1.shape[0]
    wc1p = jnp.pad(w_comp1, ((0, RPAD - nrel), (0, 0)))
    wc2p = jnp.pad(w_comp2, ((0, RPAD - nrel), (0, 0)))

    sel = jnp.repeat(jnp.eye(NBASES, dtype=jnp.float32), H,
                     axis=1).astype(jnp.bfloat16)

    h0 = _sc_gather_h0(N)(ent_embeds, nid)

    srca = srci[:EHALF]
    srcb = srci[EHALF:]

    hs1a = _sc_gather_rows(N, EHALF)(h0, srca)
    msg1a = _tc_msg(EHALF)(hs1a, et2d, wc1p, sel, wall1)
    hs1b = _sc_gather_rows(N, EHALF)(h0, srcb)
    msg1b = _tc_msg(EHALF, nblk_half)(hs1b, et2d, wc1p, sel, wall1)
    agg1 = _sc_scatter_add(N, E)(msg1a, msg1b, dsti)
    h1 = _tc_hnext(N)(agg1, h0, W_loop1)

    hs2a = _sc_gather_rows(N, EHALF)(h1, srca)
    msg2a = _tc_msg(EHALF)(hs2a, et2d, wc2p, sel, wall2)
    hs2b = _sc_gather_rows(N, EHALF)(h1, srcb)
    msg2b = _tc_msg(EHALF, nblk_half)(hs2b, et2d, wc2p, sel, wall2)
    agg2 = _sc_scatter_add(N, E)(msg2a, msg2b, dsti)

    gid2 = graph_ids.astype(jnp.int32).reshape(N, 1)
    gi = _tc_pool(N, SEQ)(agg2, h1, W_loop2, gid2)

    tid2 = time_idx.astype(jnp.int32).reshape(TB * SEQ, 1)
    emb = _tc_select(TB * SEQ, SEQ)(gi, tid2)
    return emb.reshape(TB, SEQ, H)


# speedup vs baseline: 1.0362x; 1.0362x over previous
import functools

import jax
import jax.numpy as jnp
from jax import lax
from jax.experimental import pallas as pl
from jax.experimental.pallas import tpu as pltpu
from jax.experimental.pallas import tpu_sc as plsc

H = 128
NBASES = 8
RPAD = 512
LANES = 16
NC, NS = 2, 16
NW = NC * NS
CHUNK = 80


def _sc_mesh():
    return plsc.VectorSubcoreMesh(core_axis_name="c", subcore_axis_name="s")


def _sc_gather_h0(N):
    n_chunks = N // CHUNK
    n_iters = -(-n_chunks // NW)

    @functools.partial(
        pl.kernel,
        mesh=_sc_mesh(),
        out_type=jax.ShapeDtypeStruct((N, H), jnp.float32),
        scratch_types=[
            pltpu.VMEM((CHUNK,), jnp.int32),
            pltpu.VMEM((CHUNK, H), jnp.float32),
            pltpu.SemaphoreType.DMA,
        ],
    )
    def k(ent_hbm, nid_hbm, h0_hbm, idx_v, rows_v, sem):
        wid = lax.axis_index("s") * NC + lax.axis_index("c")

        def body(kk, _):
            cidx = wid + kk * NW

            @pl.when(cidx < n_chunks)
            def _go():
                base = cidx * CHUNK
                pltpu.sync_copy(nid_hbm.at[pl.ds(base, CHUNK)], idx_v)
                pltpu.async_copy(ent_hbm.at[idx_v], rows_v, sem).wait()
                pltpu.sync_copy(rows_v, h0_hbm.at[pl.ds(base, CHUNK)])

            return _

        lax.fori_loop(0, n_iters, body, 0)

    return k


def _sc_gather_rows(N, E):
    CH = 128
    NBUF = 4
    nch = E // CH
    cpt = nch // NW
    extra = nch - cpt * NW
    ngrp = -(-(cpt + 1) // NBUF)

    @functools.partial(
        pl.kernel,
        mesh=_sc_mesh(),
        out_type=jax.ShapeDtypeStruct((E, H), jnp.float32),
        scratch_types=[
            pltpu.VMEM((cpt * CH + CH,), jnp.int32),
            [pltpu.VMEM((CH, H), jnp.float32) for _ in range(NBUF)],
            [pltpu.SemaphoreType.DMA for _ in range(NBUF)],
            [pltpu.SemaphoreType.DMA for _ in range(NBUF)],
        ],
    )
    def k(h_hbm, src_hbm, out_hbm, idx_all, rows, gsem, wsem):
        wid = lax.axis_index("s") * NC + lax.axis_index("c")
        nt = cpt + (wid < extra).astype(jnp.int32)
        pltpu.sync_copy(src_hbm.at[pl.ds(wid * (cpt * CH), cpt * CH)],
                        idx_all.at[pl.ds(0, cpt * CH)])

        @pl.when(wid < extra)
        def _extra():
            pltpu.sync_copy(src_hbm.at[pl.ds((cpt * NW + wid) * CH, CH)],
                            idx_all.at[pl.ds(cpt * CH, CH)])

        def out_base(kk):
            return jnp.where(kk < cpt, (wid * cpt + kk) * CH,
                             (cpt * NW + wid) * CH)

        for b in range(NBUF):
            pltpu.async_copy(h_hbm.at[idx_all.at[pl.ds(b * CH, CH)]],
                             rows[b], gsem[b])

        def grp(g, _):
            for b in range(NBUF):
                kk = g * NBUF + b

                @pl.when(kk < nt)
                def _do(kk=kk, b=b):
                    pltpu.make_async_copy(
                        h_hbm.at[idx_all.at[pl.ds(0, CH)]], rows[b],
                        gsem[b]).wait()
                    pltpu.async_copy(rows[b],
                                     out_hbm.at[pl.ds(out_base(kk), CH)],
                                     wsem[b])

                kn = kk + NBUF

                @pl.when(kn < nt)
                def _next(kn=kn, b=b):
                    pltpu.make_async_copy(
                        rows[b], out_hbm.at[pl.ds(0, CH)], wsem[b]).wait()
                    pltpu.async_copy(
                        h_hbm.at[idx_all.at[pl.ds(kn * CH, CH)]],
                        rows[b], gsem[b])

            return _

        lax.fori_loop(0, ngrp, grp, 0)
        for b in range(NBUF):
            pltpu.make_async_copy(rows[b], out_hbm.at[pl.ds(0, CH)],
                                  wsem[b]).wait()

    return k


def _sc_scatter_add(N, E):
    NH = N // NC
    CH = 128
    NBUF = 3
    DPAD = 8
    EH = E // 2
    nch = EH // CH
    cpt = nch // NS
    extra = nch - cpt * NS
    ngrp = -(-(cpt + 1) // NBUF)
    ZCH = 80
    n_chunks_z = NH // ZCH
    z_iters = -(-n_chunks_z // NS)

    @functools.partial(
        pl.kernel,
        mesh=_sc_mesh(),
        out_type=jax.ShapeDtypeStruct((N, H), jnp.float32),
        scratch_types=[
            pltpu.VMEM_SHARED((NH + DPAD, H), jnp.float32),
            pltpu.VMEM((NBUF, CH), jnp.int32),
            pltpu.VMEM((NBUF, CH), jnp.int32),
            [pltpu.VMEM((CH, H), jnp.float32) for _ in range(NBUF)],
            [pltpu.SemaphoreType.DMA for _ in range(NBUF)],
            [pltpu.SemaphoreType.DMA for _ in range(NBUF)],
        ],
    )
    def k(msga_hbm, msgb_hbm, dst_hbm, agg_hbm, aggS, dst2d, li2d, mbufs,
          gsem, dsem):
        c = lax.axis_index("c")
        s = lax.axis_index("s")
        node0 = c * NH
        nt = cpt + (s < extra).astype(jnp.int32)

        def msg_base(kk):
            return jnp.where(kk < cpt, (s * cpt + kk) * CH,
                             (cpt * NS + s) * CH)

        zero = jnp.zeros((LANES,), jnp.float32)

        def zb(i, _):
            for j in range(H // LANES):
                mbufs[0][i, pl.ds(j * LANES, LANES)] = zero
            return _

        lax.fori_loop(0, CH, zb, 0)

        def zs(kk, _):
            cidx = s + kk * NS

            @pl.when(cidx < n_chunks_z)
            def _go():
                pltpu.sync_copy(mbufs[0].at[pl.ds(0, ZCH)],
                                aggS.at[pl.ds(cidx * ZCH, ZCH)])

            return _

        lax.fori_loop(0, z_iters, zs, 0)

        @pl.when(s == 0)
        def _zdummy():
            pltpu.sync_copy(mbufs[0].at[pl.ds(0, DPAD)],
                            aggS.at[pl.ds(NH, DPAD)])

        plsc.subcore_barrier()

        def run_half(msg_hbm, hoff):
            def start_chunk(kk, b):
                base = msg_base(kk)
                pltpu.async_copy(dst_hbm.at[pl.ds(hoff + base, CH)],
                                 dst2d.at[b], dsem[b])
                pltpu.async_copy(msg_hbm.at[pl.ds(base, CH)], mbufs[b],
                                 gsem[b])

            for b in range(NBUF):
                start_chunk(b, b)

            def grp(g, _):
                for b in range(NBUF):
                    kk = g * NBUF + b

                    @pl.when(kk < nt)
                    def _do(kk=kk, b=b):
                        pltpu.make_async_copy(
                            dst_hbm.at[pl.ds(0, CH)], dst2d.at[b],
                            dsem[b]).wait()
                        for j in range(CH // LANES):
                            sl = pl.ds(j * LANES, LANES)
                            li = dst2d[b, sl] - node0
                            oob = (li < 0) | (li >= NH)
                            li2d[b, sl] = jnp.where(oob, NH, li)
                        pltpu.make_async_copy(
                            msg_hbm.at[pl.ds(0, CH)], mbufs[b],
                            gsem[b]).wait()
                        pltpu.sync_copy(mbufs[b], aggS.at[li2d.at[b]],
                                        add=True)

                    kn = kk + NBUF

                    @pl.when(kn < nt)
                    def _next(kn=kn, b=b):
                        start_chunk(kn, b)

                return _

            lax.fori_loop(0, ngrp, grp, 0)

        run_half(msga_hbm, 0)
        run_half(msgb_hbm, EH)
        plsc.subcore_barrier()

        def wb(kk, _):
            cidx = s + kk * NS

            @pl.when(cidx < n_chunks_z)
            def _go():
                base = cidx * ZCH
                pltpu.sync_copy(aggS.at[pl.ds(base, ZCH)],
                                mbufs[1].at[pl.ds(0, ZCH)])
                pltpu.sync_copy(mbufs[1].at[pl.ds(0, ZCH)],
                                agg_hbm.at[pl.ds(node0 + base, ZCH)])

            return _

        lax.fori_loop(0, z_iters, wb, 0)

    return k


def _tc_msg(E, blk0=0):
    EB = 3200

    def body(hs_ref, et_ref, wc_ref, sel_ref, wall_ref, msg_ref):
        g = jnp.dot(hs_ref[...].astype(jnp.bfloat16),
                    wall_ref[...].astype(jnp.bfloat16),
                    preferred_element_type=jnp.float32)
        et = et_ref[...].astype(jnp.int32)
        onehot = (et == lax.broadcasted_iota(jnp.int32, (1, RPAD), 1)
                  ).astype(jnp.bfloat16)
        coef = jnp.dot(onehot, wc_ref[...].astype(jnp.bfloat16),
                       preferred_element_type=jnp.float32)
        coefe = jnp.dot(coef.astype(jnp.bfloat16), sel_ref[...],
                        preferred_element_type=jnp.float32)
        msg = coefe[:, 0:H] * g[:, 0:H]
        for b in range(1, NBASES):
            msg = msg + coefe[:, b * H:(b + 1) * H] * g[:, b * H:(b + 1) * H]
        msg_ref[...] = msg

    return pl.pallas_call(
        body,
        grid=(E // EB,),
        in_specs=[
            pl.BlockSpec((EB, H), lambda i: (i, 0)),
            pl.BlockSpec((EB, 1), lambda i: (i + blk0, 0)),
            pl.BlockSpec((RPAD, NBASES), lambda i: (0, 0)),
            pl.BlockSpec((NBASES, NBASES * H), lambda i: (0, 0)),
            pl.BlockSpec((H, NBASES * H), lambda i: (0, 0)),
        ],
        out_specs=pl.BlockSpec((EB, H), lambda i: (i, 0)),
        out_shape=jax.ShapeDtypeStruct((E, H), jnp.float32),
    )


def _tc_hnext(N):
    NBK = 2000

    def body(agg_ref, h_ref, wl_ref, out_ref):
        out = agg_ref[...] + jnp.dot(h_ref[...], wl_ref[...],
                                     preferred_element_type=jnp.float32)
        out_ref[...] = jnp.maximum(out, 0.0)

    return pl.pallas_call(
        body,
        grid=(N // NBK,),
        in_specs=[
            pl.BlockSpec((NBK, H), lambda i: (i, 0)),
            pl.BlockSpec((NBK, H), lambda i: (i, 0)),
            pl.BlockSpec((H, H), lambda i: (0, 0)),
        ],
        out_specs=pl.BlockSpec((NBK, H), lambda i: (i, 0)),
        out_shape=jax.ShapeDtypeStruct((N, H), jnp.float32),
    )


def _tc_pool(N, SEQ):
    NBK = 2000
    nsteps = N // NBK

    def body(agg_ref, h_ref, wl_ref, gid_ref, out_ref, acc_ref):
        i = pl.program_id(0)

        @pl.when(i == 0)
        def _init():
            acc_ref[...] = jnp.full((16, H), -jnp.inf, jnp.float32)

        h2 = agg_ref[...] + jnp.dot(h_ref[...], wl_ref[...],
                                    preferred_element_type=jnp.float32)
        gid = gid_ref[...]
        for g in range(SEQ):
            cand = jnp.where(gid == g, h2, -jnp.inf)
            acc_ref[g:g + 1, :] = jnp.maximum(
                acc_ref[g:g + 1, :], jnp.max(cand, axis=0, keepdims=True))

        @pl.when(i == nsteps - 1)
        def _fin():
            out_ref[...] = acc_ref[...]

    return pl.pallas_call(
        body,
        grid=(nsteps,),
        in_specs=[
            pl.BlockSpec((NBK, H), lambda i: (i, 0)),
            pl.BlockSpec((NBK, H), lambda i: (i, 0)),
            pl.BlockSpec((H, H), lambda i: (0, 0)),
            pl.BlockSpec((NBK, 1), lambda i: (i, 0)),
        ],
        out_specs=pl.BlockSpec((16, H), lambda i: (0, 0)),
        out_shape=jax.ShapeDtypeStruct((16, H), jnp.float32),
        scratch_shapes=[pltpu.VMEM((16, H), jnp.float32)],
    )


def _tc_select(TQ, SEQ):
    def body(gi_ref, tid_ref, out_ref):
        tid = tid_ref[...]
        acc = jnp.zeros((TQ, H), jnp.float32)
        for g in range(SEQ):
            acc = jnp.where(tid == g, gi_ref[g:g + 1, :], acc)
        out_ref[...] = acc

    return pl.pallas_call(
        body,
        grid=(1,),
        in_specs=[
            pl.BlockSpec((16, H), lambda i: (0, 0)),
            pl.BlockSpec((TQ, 1), lambda i: (0, 0)),
        ],
        out_specs=pl.BlockSpec((TQ, H), lambda i: (0, 0)),
        out_shape=jax.ShapeDtypeStruct((TQ, H), jnp.float32),
    )


def kernel(ent_embeds, node_id, src, dst, edge_type, graph_ids, time_idx,
           W_basis1, w_comp1, W_loop1, W_basis2, w_comp2, W_loop2):
    N = node_id.shape[0]
    E = src.shape[0]
    TB, SEQ = time_idx.shape

    EHALF = E // 2
    nid = node_id.astype(jnp.int32)
    srci = src.astype(jnp.int32)
    dsti = dst.astype(jnp.int32)
    et2d = edge_type.astype(jnp.int16).reshape(E, 1)
    nblk_half = EHALF // 3200

    wall1 = jnp.transpose(W_basis1, (1, 0, 2)).reshape(H, NBASES * H)
    wall2 = jnp.transpose(W_basis2, (1, 0, 2)).reshape(H, NBASES * H)
    nrel = w_comp1.shape[0]
    wc1p = jnp.pad(w_comp1, ((0, RPAD - nrel), (0, 0)))
    wc2p = jnp.pad(w_comp2, ((0, RPAD - nrel), (0, 0)))

    sel = jnp.repeat(jnp.eye(NBASES, dtype=jnp.float32), H,
                     axis=1).astype(jnp.bfloat16)

    h0 = _sc_gather_h0(N)(ent_embeds, nid)

    srca = srci[:EHALF]
    srcb = srci[EHALF:]

    hs1a = _sc_gather_rows(N, EHALF)(h0, srca)
    msg1a = _tc_msg(EHALF)(hs1a, et2d, wc1p, sel, wall1)
    hs1b = _sc_gather_rows(N, EHALF)(h0, srcb)
    msg1b = _tc_msg(EHALF, nblk_half)(hs1b, et2d, wc1p, sel, wall1)
    agg1 = _sc_scatter_add(N, E)(msg1a, msg1b, dsti)
    h1 = _tc_hnext(N)(agg1, h0, W_loop1)

    hs2a = _sc_gather_rows(N, EHALF)(h1, srca)
    msg2a = _tc_msg(EHALF)(hs2a, et2d, wc2p, sel, wall2)
    hs2b = _sc_gather_rows(N, EHALF)(h1, srcb)
    msg2b = _tc_msg(EHALF, nblk_half)(hs2b, et2d, wc2p, sel, wall2)
    agg2 = _sc_scatter_add(N, E)(msg2a, msg2b, dsti)

    gid2 = graph_ids.astype(jnp.int32).reshape(N, 1)
    gi = _tc_pool(N, SEQ)(agg2, h1, W_loop2, gid2)

    tid2 = time_idx.astype(jnp.int32).reshape(TB * SEQ, 1)
    emb = _tc_select(TB * SEQ, SEQ)(gi, tid2)
    return emb.reshape(TB, SEQ, H)


# speedup vs baseline: 1.0433x; 1.0069x over previous
import functools

import jax
import jax.numpy as jnp
from jax import lax
from jax.experimental import pallas as pl
from jax.experimental.pallas import tpu as pltpu
from jax.experimental.pallas import tpu_sc as plsc

H = 128
NBASES = 8
RPAD = 512
LANES = 16
NC, NS = 2, 16
NW = NC * NS
CHUNK = 80


def _sc_mesh():
    return plsc.VectorSubcoreMesh(core_axis_name="c", subcore_axis_name="s")


def _sc_gather_h0(N):
    n_chunks = N // CHUNK
    n_iters = -(-n_chunks // NW)

    @functools.partial(
        pl.kernel,
        mesh=_sc_mesh(),
        out_type=jax.ShapeDtypeStruct((N, H), jnp.float32),
        scratch_types=[
            pltpu.VMEM((CHUNK,), jnp.int32),
            pltpu.VMEM((CHUNK, H), jnp.float32),
            pltpu.SemaphoreType.DMA,
        ],
    )
    def k(ent_hbm, nid_hbm, h0_hbm, idx_v, rows_v, sem):
        wid = lax.axis_index("s") * NC + lax.axis_index("c")

        def body(kk, _):
            cidx = wid + kk * NW

            @pl.when(cidx < n_chunks)
            def _go():
                base = cidx * CHUNK
                pltpu.sync_copy(nid_hbm.at[pl.ds(base, CHUNK)], idx_v)
                pltpu.async_copy(ent_hbm.at[idx_v], rows_v, sem).wait()
                pltpu.sync_copy(rows_v, h0_hbm.at[pl.ds(base, CHUNK)])

            return _

        lax.fori_loop(0, n_iters, body, 0)

    return k


def _sc_gather_rows(N, E):
    CH = 128
    NBUF = 4
    nch = E // CH
    cpt = nch // NW
    extra = nch - cpt * NW
    ngrp = -(-(cpt + 1) // NBUF)

    @functools.partial(
        pl.kernel,
        mesh=_sc_mesh(),
        out_type=jax.ShapeDtypeStruct((E, H), jnp.float32),
        scratch_types=[
            pltpu.VMEM((cpt * CH + CH,), jnp.int32),
            [pltpu.VMEM((CH, H), jnp.float32) for _ in range(NBUF)],
            [pltpu.SemaphoreType.DMA for _ in range(NBUF)],
            [pltpu.SemaphoreType.DMA for _ in range(NBUF)],
        ],
    )
    def k(h_hbm, src_hbm, out_hbm, idx_all, rows, gsem, wsem):
        wid = lax.axis_index("s") * NC + lax.axis_index("c")
        nt = cpt + (wid < extra).astype(jnp.int32)
        pltpu.sync_copy(src_hbm.at[pl.ds(wid * (cpt * CH), cpt * CH)],
                        idx_all.at[pl.ds(0, cpt * CH)])

        @pl.when(wid < extra)
        def _extra():
            pltpu.sync_copy(src_hbm.at[pl.ds((cpt * NW + wid) * CH, CH)],
                            idx_all.at[pl.ds(cpt * CH, CH)])

        def out_base(kk):
            return jnp.where(kk < cpt, (wid * cpt + kk) * CH,
                             (cpt * NW + wid) * CH)

        for b in range(NBUF):
            pltpu.async_copy(h_hbm.at[idx_all.at[pl.ds(b * CH, CH)]],
                             rows[b], gsem[b])

        def grp(g, _):
            for b in range(NBUF):
                kk = g * NBUF + b

                @pl.when(kk < nt)
                def _do(kk=kk, b=b):
                    pltpu.make_async_copy(
                        h_hbm.at[idx_all.at[pl.ds(0, CH)]], rows[b],
                        gsem[b]).wait()
                    pltpu.async_copy(rows[b],
                                     out_hbm.at[pl.ds(out_base(kk), CH)],
                                     wsem[b])

                kn = kk + NBUF

                @pl.when(kn < nt)
                def _next(kn=kn, b=b):
                    pltpu.make_async_copy(
                        rows[b], out_hbm.at[pl.ds(0, CH)], wsem[b]).wait()
                    pltpu.async_copy(
                        h_hbm.at[idx_all.at[pl.ds(kn * CH, CH)]],
                        rows[b], gsem[b])

            return _

        lax.fori_loop(0, ngrp, grp, 0)
        for b in range(NBUF):
            pltpu.make_async_copy(rows[b], out_hbm.at[pl.ds(0, CH)],
                                  wsem[b]).wait()

    return k


def _sc_scatter_add(N, E):
    NH = N // NC
    CH = 128
    NBUF = 3
    DPAD = 8
    EH = E // 2
    nch = EH // CH
    cpt = nch // NS
    extra = nch - cpt * NS
    ngrp = -(-(cpt + 1) // NBUF)
    ZCH = 80
    n_chunks_z = NH // ZCH
    z_iters = -(-n_chunks_z // NS)

    @functools.partial(
        pl.kernel,
        mesh=_sc_mesh(),
        out_type=jax.ShapeDtypeStruct((N, H), jnp.float32),
        scratch_types=[
            pltpu.VMEM_SHARED((NH + DPAD, H), jnp.float32),
            pltpu.VMEM((NBUF, CH), jnp.int32),
            pltpu.VMEM((NBUF, CH), jnp.int32),
            [pltpu.VMEM((CH, H), jnp.float32) for _ in range(NBUF)],
            [pltpu.SemaphoreType.DMA for _ in range(NBUF)],
            [pltpu.SemaphoreType.DMA for _ in range(NBUF)],
        ],
    )
    def k(msga_hbm, msgb_hbm, dst_hbm, agg_hbm, aggS, dst2d, li2d, mbufs,
          gsem, dsem):
        c = lax.axis_index("c")
        s = lax.axis_index("s")
        node0 = c * NH
        nt = cpt + (s < extra).astype(jnp.int32)

        def msg_base(kk):
            return jnp.where(kk < cpt, (s * cpt + kk) * CH,
                             (cpt * NS + s) * CH)

        zero = jnp.zeros((LANES,), jnp.float32)

        def zb(i, _):
            for j in range(H // LANES):
                mbufs[0][i, pl.ds(j * LANES, LANES)] = zero
            return _

        lax.fori_loop(0, CH, zb, 0)

        def zs(kk, _):
            cidx = s + kk * NS

            @pl.when(cidx < n_chunks_z)
            def _go():
                pltpu.sync_copy(mbufs[0].at[pl.ds(0, ZCH)],
                                aggS.at[pl.ds(cidx * ZCH, ZCH)])

            return _

        lax.fori_loop(0, z_iters, zs, 0)

        @pl.when(s == 0)
        def _zdummy():
            pltpu.sync_copy(mbufs[0].at[pl.ds(0, DPAD)],
                            aggS.at[pl.ds(NH, DPAD)])

        plsc.subcore_barrier()

        def run_half(msg_hbm, hoff):
            def start_chunk(kk, b):
                base = msg_base(kk)
                pltpu.async_copy(dst_hbm.at[pl.ds(hoff + base, CH)],
                                 dst2d.at[b], dsem[b])
                pltpu.async_copy(msg_hbm.at[pl.ds(base, CH)], mbufs[b],
                                 gsem[b])

            for b in range(NBUF):
                start_chunk(b, b)

            def grp(g, _):
                for b in range(NBUF):
                    kk = g * NBUF + b

                    @pl.when(kk < nt)
                    def _do(kk=kk, b=b):
                        pltpu.make_async_copy(
                            dst_hbm.at[pl.ds(0, CH)], dst2d.at[b],
                            dsem[b]).wait()
                        for j in range(CH // LANES):
                            sl = pl.ds(j * LANES, LANES)
                            li = dst2d[b, sl] - node0
                            oob = (li < 0) | (li >= NH)
                            li2d[b, sl] = jnp.where(oob, NH, li)
                        pltpu.make_async_copy(
                            msg_hbm.at[pl.ds(0, CH)], mbufs[b],
                            gsem[b]).wait()
                        pltpu.sync_copy(mbufs[b], aggS.at[li2d.at[b]],
                                        add=True)

                    kn = kk + NBUF

                    @pl.when(kn < nt)
                    def _next(kn=kn, b=b):
                        start_chunk(kn, b)

                return _

            lax.fori_loop(0, ngrp, grp, 0)

        run_half(msga_hbm, 0)
        run_half(msgb_hbm, EH)
        plsc.subcore_barrier()

        def wb(kk, _):
            cidx = s + kk * NS

            @pl.when(cidx < n_chunks_z)
            def _go():
                base = cidx * ZCH
                pltpu.sync_copy(aggS.at[pl.ds(base, ZCH)],
                                mbufs[1].at[pl.ds(0, ZCH)])
                pltpu.sync_copy(mbufs[1].at[pl.ds(0, ZCH)],
                                agg_hbm.at[pl.ds(node0 + base, ZCH)])

            return _

        lax.fori_loop(0, z_iters, wb, 0)

    return k


def _tc_msg(E, blk0=0):
    EB = 4000

    def body(hs_ref, et_ref, wc_ref, sel_ref, wall_ref, msg_ref):
        g = jnp.dot(hs_ref[...].astype(jnp.bfloat16),
                    wall_ref[...].astype(jnp.bfloat16),
                    preferred_element_type=jnp.float32)
        et = et_ref[...].astype(jnp.int32)
        onehot = (et == lax.broadcasted_iota(jnp.int32, (1, RPAD), 1)
                  ).astype(jnp.bfloat16)
        coef = jnp.dot(onehot, wc_ref[...].astype(jnp.bfloat16),
                       preferred_element_type=jnp.float32)
        coefe = jnp.dot(coef.astype(jnp.bfloat16), sel_ref[...],
                        preferred_element_type=jnp.float32)
        msg = coefe[:, 0:H] * g[:, 0:H]
        for b in range(1, NBASES):
            msg = msg + coefe[:, b * H:(b + 1) * H] * g[:, b * H:(b + 1) * H]
        msg_ref[...] = msg

    return pl.pallas_call(
        body,
        grid=(E // EB,),
        in_specs=[
            pl.BlockSpec((EB, H), lambda i: (i, 0)),
            pl.BlockSpec((EB, 1), lambda i: (i + blk0, 0)),
            pl.BlockSpec((RPAD, NBASES), lambda i: (0, 0)),
            pl.BlockSpec((NBASES, NBASES * H), lambda i: (0, 0)),
            pl.BlockSpec((H, NBASES * H), lambda i: (0, 0)),
        ],
        out_specs=pl.BlockSpec((EB, H), lambda i: (i, 0)),
        out_shape=jax.ShapeDtypeStruct((E, H), jnp.float32),
    )


def _tc_hnext(N):
    NBK = 2000

    def body(agg_ref, h_ref, wl_ref, out_ref):
        out = agg_ref[...] + jnp.dot(h_ref[...], wl_ref[...],
                                     preferred_element_type=jnp.float32)
        out_ref[...] = jnp.maximum(out, 0.0)

    return pl.pallas_call(
        body,
        grid=(N // NBK,),
        in_specs=[
            pl.BlockSpec((NBK, H), lambda i: (i, 0)),
            pl.BlockSpec((NBK, H), lambda i: (i, 0)),
            pl.BlockSpec((H, H), lambda i: (0, 0)),
        ],
        out_specs=pl.BlockSpec((NBK, H), lambda i: (i, 0)),
        out_shape=jax.ShapeDtypeStruct((N, H), jnp.float32),
    )


def _tc_pool(N, SEQ):
    NBK = 2000
    nsteps = N // NBK

    def body(agg_ref, h_ref, wl_ref, gid_ref, out_ref, acc_ref):
        i = pl.program_id(0)

        @pl.when(i == 0)
        def _init():
            acc_ref[...] = jnp.full((16, H), -jnp.inf, jnp.float32)

        h2 = agg_ref[...] + jnp.dot(h_ref[...], wl_ref[...],
                                    preferred_element_type=jnp.float32)
        gid = gid_ref[...]
        for g in range(SEQ):
            cand = jnp.where(gid == g, h2, -jnp.inf)
            acc_ref[g:g + 1, :] = jnp.maximum(
                acc_ref[g:g + 1, :], jnp.max(cand, axis=0, keepdims=True))

        @pl.when(i == nsteps - 1)
        def _fin():
            out_ref[...] = acc_ref[...]

    return pl.pallas_call(
        body,
        grid=(nsteps,),
        in_specs=[
            pl.BlockSpec((NBK, H), lambda i: (i, 0)),
            pl.BlockSpec((NBK, H), lambda i: (i, 0)),
            pl.BlockSpec((H, H), lambda i: (0, 0)),
            pl.BlockSpec((NBK, 1), lambda i: (i, 0)),
        ],
        out_specs=pl.BlockSpec((16, H), lambda i: (0, 0)),
        out_shape=jax.ShapeDtypeStruct((16, H), jnp.float32),
        scratch_shapes=[pltpu.VMEM((16, H), jnp.float32)],
    )


def _tc_select(TQ, SEQ):
    def body(gi_ref, tid_ref, out_ref):
        tid = tid_ref[...]
        acc = jnp.zeros((TQ, H), jnp.float32)
        for g in range(SEQ):
            acc = jnp.where(tid == g, gi_ref[g:g + 1, :], acc)
        out_ref[...] = acc

    return pl.pallas_call(
        body,
        grid=(1,),
        in_specs=[
            pl.BlockSpec((16, H), lambda i: (0, 0)),
            pl.BlockSpec((TQ, 1), lambda i: (0, 0)),
        ],
        out_specs=pl.BlockSpec((TQ, H), lambda i: (0, 0)),
        out_shape=jax.ShapeDtypeStruct((TQ, H), jnp.float32),
    )


def kernel(ent_embeds, node_id, src, dst, edge_type, graph_ids, time_idx,
           W_basis1, w_comp1, W_loop1, W_basis2, w_comp2, W_loop2):
    N = node_id.shape[0]
    E = src.shape[0]
    TB, SEQ = time_idx.shape

    EHALF = E // 2
    nid = node_id.astype(jnp.int32)
    srci = src.astype(jnp.int32)
    dsti = dst.astype(jnp.int32)
    et2d = edge_type.astype(jnp.int16).reshape(E, 1)
    nblk_half = EHALF // 4000

    wall1 = jnp.transpose(W_basis1, (1, 0, 2)).reshape(H, NBASES * H)
    wall2 = jnp.transpose(W_basis2, (1, 0, 2)).reshape(H, NBASES * H)
    nrel = w_comp1.shape[0]
    wc1p = jnp.pad(w_comp1, ((0, RPAD - nrel), (0, 0)))
    wc2p = jnp.pad(w_comp2, ((0, RPAD - nrel), (0, 0)))

    sel = jnp.repeat(jnp.eye(NBASES, dtype=jnp.float32), H,
                     axis=1).astype(jnp.bfloat16)

    h0 = _sc_gather_h0(N)(ent_embeds, nid)

    srca = srci[:EHALF]
    srcb = srci[EHALF:]

    hs1a = _sc_gather_rows(N, EHALF)(h0, srca)
    msg1a = _tc_msg(EHALF)(hs1a, et2d, wc1p, sel, wall1)
    hs1b = _sc_gather_rows(N, EHALF)(h0, srcb)
    msg1b = _tc_msg(EHALF, nblk_half)(hs1b, et2d, wc1p, sel, wall1)
    agg1 = _sc_scatter_add(N, E)(msg1a, msg1b, dsti)
    h1 = _tc_hnext(N)(agg1, h0, W_loop1)

    hs2a = _sc_gather_rows(N, EHALF)(h1, srca)
    msg2a = _tc_msg(EHALF)(hs2a, et2d, wc2p, sel, wall2)
    hs2b = _sc_gather_rows(N, EHALF)(h1, srcb)
    msg2b = _tc_msg(EHALF, nblk_half)(hs2b, et2d, wc2p, sel, wall2)
    agg2 = _sc_scatter_add(N, E)(msg2a, msg2b, dsti)

    gid2 = graph_ids.astype(jnp.int32).reshape(N, 1)
    gi = _tc_pool(N, SEQ)(agg2, h1, W_loop2, gid2)

    tid2 = time_idx.astype(jnp.int32).reshape(TB * SEQ, 1)
    emb = _tc_select(TB * SEQ, SEQ)(gi, tid2)
    return emb.reshape(TB, SEQ, H)


# speedup vs baseline: 1.0487x; 1.0052x over previous
import functools

import jax
import jax.numpy as jnp
from jax import lax
from jax.experimental import pallas as pl
from jax.experimental.pallas import tpu as pltpu
from jax.experimental.pallas import tpu_sc as plsc

H = 128
NBASES = 8
RPAD = 512
LANES = 16
NC, NS = 2, 16
NW = NC * NS
CHUNK = 80


def _sc_mesh():
    return plsc.VectorSubcoreMesh(core_axis_name="c", subcore_axis_name="s")


def _sc_gather_h0(N):
    n_chunks = N // CHUNK
    n_iters = -(-n_chunks // NW)

    @functools.partial(
        pl.kernel,
        mesh=_sc_mesh(),
        out_type=jax.ShapeDtypeStruct((N, H), jnp.float32),
        scratch_types=[
            pltpu.VMEM((CHUNK,), jnp.int32),
            pltpu.VMEM((CHUNK, H), jnp.float32),
            pltpu.SemaphoreType.DMA,
        ],
    )
    def k(ent_hbm, nid_hbm, h0_hbm, idx_v, rows_v, sem):
        wid = lax.axis_index("s") * NC + lax.axis_index("c")

        def body(kk, _):
            cidx = wid + kk * NW

            @pl.when(cidx < n_chunks)
            def _go():
                base = cidx * CHUNK
                pltpu.sync_copy(nid_hbm.at[pl.ds(base, CHUNK)], idx_v)
                pltpu.async_copy(ent_hbm.at[idx_v], rows_v, sem).wait()
                pltpu.sync_copy(rows_v, h0_hbm.at[pl.ds(base, CHUNK)])

            return _

        lax.fori_loop(0, n_iters, body, 0)

    return k


def _sc_gather_rows(N, E):
    CH = 128
    NBUF = 4
    nch = E // CH
    cpt = nch // NW
    extra = nch - cpt * NW
    ngrp = -(-(cpt + 1) // NBUF)

    @functools.partial(
        pl.kernel,
        mesh=_sc_mesh(),
        out_type=jax.ShapeDtypeStruct((E, H), jnp.float32),
        scratch_types=[
            pltpu.VMEM((cpt * CH + CH,), jnp.int32),
            [pltpu.VMEM((CH, H), jnp.float32) for _ in range(NBUF)],
            [pltpu.SemaphoreType.DMA for _ in range(NBUF)],
            [pltpu.SemaphoreType.DMA for _ in range(NBUF)],
        ],
    )
    def k(h_hbm, src_hbm, out_hbm, idx_all, rows, gsem, wsem):
        wid = lax.axis_index("s") * NC + lax.axis_index("c")
        nt = cpt + (wid < extra).astype(jnp.int32)
        pltpu.sync_copy(src_hbm.at[pl.ds(wid * (cpt * CH), cpt * CH)],
                        idx_all.at[pl.ds(0, cpt * CH)])

        @pl.when(wid < extra)
        def _extra():
            pltpu.sync_copy(src_hbm.at[pl.ds((cpt * NW + wid) * CH, CH)],
                            idx_all.at[pl.ds(cpt * CH, CH)])

        def out_base(kk):
            return jnp.where(kk < cpt, (wid * cpt + kk) * CH,
                             (cpt * NW + wid) * CH)

        for b in range(NBUF):
            pltpu.async_copy(h_hbm.at[idx_all.at[pl.ds(b * CH, CH)]],
                             rows[b], gsem[b])

        def grp(g, _):
            for b in range(NBUF):
                kk = g * NBUF + b

                @pl.when(kk < nt)
                def _do(kk=kk, b=b):
                    pltpu.make_async_copy(
                        h_hbm.at[idx_all.at[pl.ds(0, CH)]], rows[b],
                        gsem[b]).wait()
                    pltpu.async_copy(rows[b],
                                     out_hbm.at[pl.ds(out_base(kk), CH)],
                                     wsem[b])

                kn = kk + NBUF

                @pl.when(kn < nt)
                def _next(kn=kn, b=b):
                    pltpu.make_async_copy(
                        rows[b], out_hbm.at[pl.ds(0, CH)], wsem[b]).wait()
                    pltpu.async_copy(
                        h_hbm.at[idx_all.at[pl.ds(kn * CH, CH)]],
                        rows[b], gsem[b])

            return _

        lax.fori_loop(0, ngrp, grp, 0)
        for b in range(NBUF):
            pltpu.make_async_copy(rows[b], out_hbm.at[pl.ds(0, CH)],
                                  wsem[b]).wait()

    return k


def _sc_scatter_add(N, E):
    NH = N // NC
    CH = 128
    NBUF = 3
    DPAD = 8
    EH = E // 2
    nch = EH // CH
    cpt = nch // NS
    extra = nch - cpt * NS
    ngrp = -(-(cpt + 1) // NBUF)
    ZCH = 80
    n_chunks_z = NH // ZCH
    z_iters = -(-n_chunks_z // NS)

    @functools.partial(
        pl.kernel,
        mesh=_sc_mesh(),
        out_type=jax.ShapeDtypeStruct((N, H), jnp.float32),
        scratch_types=[
            pltpu.VMEM_SHARED((NH + DPAD, H), jnp.float32),
            pltpu.VMEM((NBUF, CH), jnp.int32),
            pltpu.VMEM((NBUF, CH), jnp.int32),
            [pltpu.VMEM((CH, H), jnp.float32) for _ in range(NBUF)],
            [pltpu.SemaphoreType.DMA for _ in range(NBUF)],
            [pltpu.SemaphoreType.DMA for _ in range(NBUF)],
        ],
    )
    def k(msga_hbm, msgb_hbm, dst_hbm, agg_hbm, aggS, dst2d, li2d, mbufs,
          gsem, dsem):
        c = lax.axis_index("c")
        s = lax.axis_index("s")
        node0 = c * NH
        nt = cpt + (s < extra).astype(jnp.int32)

        def msg_base(kk):
            return jnp.where(kk < cpt, (s * cpt + kk) * CH,
                             (cpt * NS + s) * CH)

        zero = jnp.zeros((LANES,), jnp.float32)

        def zb(i, _):
            for j in range(H // LANES):
                mbufs[0][i, pl.ds(j * LANES, LANES)] = zero
            return _

        lax.fori_loop(0, CH, zb, 0)

        def zs(kk, _):
            cidx = s + kk * NS

            @pl.when(cidx < n_chunks_z)
            def _go():
                pltpu.sync_copy(mbufs[0].at[pl.ds(0, ZCH)],
                                aggS.at[pl.ds(cidx * ZCH, ZCH)])

            return _

        lax.fori_loop(0, z_iters, zs, 0)

        @pl.when(s == 0)
        def _zdummy():
            pltpu.sync_copy(mbufs[0].at[pl.ds(0, DPAD)],
                            aggS.at[pl.ds(NH, DPAD)])

        plsc.subcore_barrier()

        def run_half(msg_hbm, hoff):
            def start_chunk(kk, b):
                base = msg_base(kk)
                pltpu.async_copy(dst_hbm.at[pl.ds(hoff + base, CH)],
                                 dst2d.at[b], dsem[b])
                pltpu.async_copy(msg_hbm.at[pl.ds(base, CH)], mbufs[b],
                                 gsem[b])

            for b in range(NBUF):
                start_chunk(b, b)

            def grp(g, _):
                for b in range(NBUF):
                    kk = g * NBUF + b

                    @pl.when(kk < nt)
                    def _do(kk=kk, b=b):
                        pltpu.make_async_copy(
                            dst_hbm.at[pl.ds(0, CH)], dst2d.at[b],
                            dsem[b]).wait()
                        for j in range(CH // LANES):
                            sl = pl.ds(j * LANES, LANES)
                            li = dst2d[b, sl] - node0
                            oob = (li < 0) | (li >= NH)
                            li2d[b, sl] = jnp.where(oob, NH, li)
                        pltpu.make_async_copy(
                            msg_hbm.at[pl.ds(0, CH)], mbufs[b],
                            gsem[b]).wait()
                        pltpu.sync_copy(mbufs[b], aggS.at[li2d.at[b]],
                                        add=True)

                    kn = kk + NBUF

                    @pl.when(kn < nt)
                    def _next(kn=kn, b=b):
                        start_chunk(kn, b)

                return _

            lax.fori_loop(0, ngrp, grp, 0)

        run_half(msga_hbm, 0)
        run_half(msgb_hbm, EH)
        plsc.subcore_barrier()

        def wb(kk, _):
            cidx = s + kk * NS

            @pl.when(cidx < n_chunks_z)
            def _go():
                base = cidx * ZCH
                pltpu.sync_copy(aggS.at[pl.ds(base, ZCH)],
                                mbufs[1].at[pl.ds(0, ZCH)])
                pltpu.sync_copy(mbufs[1].at[pl.ds(0, ZCH)],
                                agg_hbm.at[pl.ds(node0 + base, ZCH)])

            return _

        lax.fori_loop(0, z_iters, wb, 0)

    return k


def _tc_msg(E, blk0=0):
    EB = 5000

    def body(hs_ref, et_ref, wc_ref, sel_ref, wall_ref, msg_ref):
        g = jnp.dot(hs_ref[...].astype(jnp.bfloat16),
                    wall_ref[...].astype(jnp.bfloat16),
                    preferred_element_type=jnp.float32)
        et = et_ref[...].astype(jnp.int32)
        onehot = (et == lax.broadcasted_iota(jnp.int32, (1, RPAD), 1)
                  ).astype(jnp.bfloat16)
        coef = jnp.dot(onehot, wc_ref[...].astype(jnp.bfloat16),
                       preferred_element_type=jnp.float32)
        coefe = jnp.dot(coef.astype(jnp.bfloat16), sel_ref[...],
                        preferred_element_type=jnp.float32)
        msg = coefe[:, 0:H] * g[:, 0:H]
        for b in range(1, NBASES):
            msg = msg + coefe[:, b * H:(b + 1) * H] * g[:, b * H:(b + 1) * H]
        msg_ref[...] = msg

    return pl.pallas_call(
        body,
        grid=(E // EB,),
        in_specs=[
            pl.BlockSpec((EB, H), lambda i: (i, 0)),
            pl.BlockSpec((EB, 1), lambda i: (i + blk0, 0)),
            pl.BlockSpec((RPAD, NBASES), lambda i: (0, 0)),
            pl.BlockSpec((NBASES, NBASES * H), lambda i: (0, 0)),
            pl.BlockSpec((H, NBASES * H), lambda i: (0, 0)),
        ],
        out_specs=pl.BlockSpec((EB, H), lambda i: (i, 0)),
        out_shape=jax.ShapeDtypeStruct((E, H), jnp.float32),
    )


def _tc_hnext(N):
    NBK = 2000

    def body(agg_ref, h_ref, wl_ref, out_ref):
        out = agg_ref[...] + jnp.dot(h_ref[...], wl_ref[...],
                                     preferred_element_type=jnp.float32)
        out_ref[...] = jnp.maximum(out, 0.0)

    return pl.pallas_call(
        body,
        grid=(N // NBK,),
        in_specs=[
            pl.BlockSpec((NBK, H), lambda i: (i, 0)),
            pl.BlockSpec((NBK, H), lambda i: (i, 0)),
            pl.BlockSpec((H, H), lambda i: (0, 0)),
        ],
        out_specs=pl.BlockSpec((NBK, H), lambda i: (i, 0)),
        out_shape=jax.ShapeDtypeStruct((N, H), jnp.float32),
    )


def _tc_pool(N, SEQ):
    NBK = 2000
    nsteps = N // NBK

    def body(agg_ref, h_ref, wl_ref, gid_ref, out_ref, acc_ref):
        i = pl.program_id(0)

        @pl.when(i == 0)
        def _init():
            acc_ref[...] = jnp.full((16, H), -jnp.inf, jnp.float32)

        h2 = agg_ref[...] + jnp.dot(h_ref[...], wl_ref[...],
                                    preferred_element_type=jnp.float32)
        gid = gid_ref[...]
        for g in range(SEQ):
            cand = jnp.where(gid == g, h2, -jnp.inf)
            acc_ref[g:g + 1, :] = jnp.maximum(
                acc_ref[g:g + 1, :], jnp.max(cand, axis=0, keepdims=True))

        @pl.when(i == nsteps - 1)
        def _fin():
            out_ref[...] = acc_ref[...]

    return pl.pallas_call(
        body,
        grid=(nsteps,),
        in_specs=[
            pl.BlockSpec((NBK, H), lambda i: (i, 0)),
            pl.BlockSpec((NBK, H), lambda i: (i, 0)),
            pl.BlockSpec((H, H), lambda i: (0, 0)),
            pl.BlockSpec((NBK, 1), lambda i: (i, 0)),
        ],
        out_specs=pl.BlockSpec((16, H), lambda i: (0, 0)),
        out_shape=jax.ShapeDtypeStruct((16, H), jnp.float32),
        scratch_shapes=[pltpu.VMEM((16, H), jnp.float32)],
    )


def _tc_select(TQ, SEQ):
    def body(gi_ref, tid_ref, out_ref):
        tid = tid_ref[...]
        acc = jnp.zeros((TQ, H), jnp.float32)
        for g in range(SEQ):
            acc = jnp.where(tid == g, gi_ref[g:g + 1, :], acc)
        out_ref[...] = acc

    return pl.pallas_call(
        body,
        grid=(1,),
        in_specs=[
            pl.BlockSpec((16, H), lambda i: (0, 0)),
            pl.BlockSpec((TQ, 1), lambda i: (0, 0)),
        ],
        out_specs=pl.BlockSpec((TQ, H), lambda i: (0, 0)),
        out_shape=jax.ShapeDtypeStruct((TQ, H), jnp.float32),
    )


def kernel(ent_embeds, node_id, src, dst, edge_type, graph_ids, time_idx,
           W_basis1, w_comp1, W_loop1, W_basis2, w_comp2, W_loop2):
    N = node_id.shape[0]
    E = src.shape[0]
    TB, SEQ = time_idx.shape

    EHALF = E // 2
    nid = node_id.astype(jnp.int32)
    srci = src.astype(jnp.int32)
    dsti = dst.astype(jnp.int32)
    et2d = edge_type.astype(jnp.int16).reshape(E, 1)
    nblk_half = EHALF // 5000

    wall1 = jnp.transpose(W_basis1, (1, 0, 2)).reshape(H, NBASES * H)
    wall2 = jnp.transpose(W_basis2, (1, 0, 2)).reshape(H, NBASES * H)
    nrel = w_comp1.shape[0]
    wc1p = jnp.pad(w_comp1, ((0, RPAD - nrel), (0, 0)))
    wc2p = jnp.pad(w_comp2, ((0, RPAD - nrel), (0, 0)))

    sel = jnp.repeat(jnp.eye(NBASES, dtype=jnp.float32), H,
                     axis=1).astype(jnp.bfloat16)

    h0 = _sc_gather_h0(N)(ent_embeds, nid)

    srca = srci[:EHALF]
    srcb = srci[EHALF:]

    hs1a = _sc_gather_rows(N, EHALF)(h0, srca)
    msg1a = _tc_msg(EHALF)(hs1a, et2d, wc1p, sel, wall1)
    hs1b = _sc_gather_rows(N, EHALF)(h0, srcb)
    msg1b = _tc_msg(EHALF, nblk_half)(hs1b, et2d, wc1p, sel, wall1)
    agg1 = _sc_scatter_add(N, E)(msg1a, msg1b, dsti)
    h1 = _tc_hnext(N)(agg1, h0, W_loop1)

    hs2a = _sc_gather_rows(N, EHALF)(h1, srca)
    msg2a = _tc_msg(EHALF)(hs2a, et2d, wc2p, sel, wall2)
    hs2b = _sc_gather_rows(N, EHALF)(h1, srcb)
    msg2b = _tc_msg(EHALF, nblk_half)(hs2b, et2d, wc2p, sel, wall2)
    agg2 = _sc_scatter_add(N, E)(msg2a, msg2b, dsti)

    gid2 = graph_ids.astype(jnp.int32).reshape(N, 1)
    gi = _tc_pool(N, SEQ)(agg2, h1, W_loop2, gid2)

    tid2 = time_idx.astype(jnp.int32).reshape(TB * SEQ, 1)
    emb = _tc_select(TB * SEQ, SEQ)(gi, tid2)
    return emb.reshape(TB, SEQ, H)
